# baseline (speedup 1.0000x reference)
import functools

import jax
import jax.numpy as jnp
from jax import lax
from jax.experimental import pallas as pl
from jax.experimental.pallas import tpu as pltpu
from jax.experimental.pallas import tpu_sc as plsc

B = 50000
S = 10
D = 128
SEG = S + 1
G = 8
NW = 32
NGRP = 196
PER_W = NGRP * G
BP = NW * PER_W
L = 16
NBUF = 4
NITER = NGRP // NBUF

_mesh = plsc.VectorSubcoreMesh(core_axis_name="c", subcore_axis_name="s")


def _mm_body(x_ref, w_ref, o_ref):
    o_ref[...] = jnp.dot(x_ref[...], w_ref[...],
                         preferred_element_type=jnp.float32)


def _transform_table(features, Wc):
    blk = 2000
    return pl.pallas_call(
        _mm_body,
        grid=(B // blk,),
        in_specs=[
            pl.BlockSpec((blk, D), lambda i: (i, 0)),
            pl.BlockSpec((D, 2 * D), lambda i: (0, 0)),
        ],
        out_specs=pl.BlockSpec((blk, 2 * D), lambda i: (i, 0)),
        out_shape=jax.ShapeDtypeStruct((B, 2 * D), jnp.float32),
    )(features, Wc)


def _reduce_group(rows_v, out_v):
    for dd in range(G):
        for j in range(D // L):
            sl = pl.ds(j * L, L)
            acc = rows_v[dd * SEG, sl]
            for t in range(1, SEG):
                acc = acc + rows_v[dd * SEG + t, sl]
            out_v[dd, sl] = acc


@functools.partial(
    pl.kernel,
    mesh=_mesh,
    out_type=jax.ShapeDtypeStruct((BP, D), jnp.float32),
    scratch_types=[
        pltpu.VMEM((NGRP, SEG * G), jnp.int32),
        pltpu.VMEM((G, D), jnp.float32),
    ] + [pltpu.VMEM((SEG * G, D), jnp.float32) for _ in range(NBUF)]
      + [pltpu.SemaphoreType.DMA for _ in range(NBUF)],
)
def _gather_reduce(table_hbm, idx_hbm, out_hbm, idx_v, out_v, *bufsem):
    bufs, sems = bufsem[:NBUF], bufsem[NBUF:]
    wid = lax.axis_index("s") * 2 + lax.axis_index("c")
    base = wid * PER_W

    pltpu.sync_copy(idx_hbm.at[wid], idx_v)
    for l in range(NBUF):
        pltpu.async_copy(table_hbm.at[idx_v.at[l]], bufs[l], sems[l])

    def body(gq, carry):
        for l in range(NBUF):
            g = NBUF * gq + l
            pltpu.make_async_copy(table_hbm.at[idx_v.at[0]],
                                  bufs[l], sems[l]).wait()
            _reduce_group(bufs[l], out_v)
            pltpu.sync_copy(out_v, out_hbm.at[pl.ds(base + g * G, G)])

            @pl.when(gq < NITER - 1)
            def _():
                pltpu.async_copy(table_hbm.at[idx_v.at[g + NBUF]],
                                 bufs[l], sems[l])
        return carry

    lax.fori_loop(0, NITER, body, 0)


def kernel(features, node_list, neigh_indices, W):
    Wc = jnp.concatenate([W[:, :D].T, W[:, D:].T * jnp.float32(1.0 / S)],
                         axis=1)
    ft = _transform_table(features, Wc)
    table = ft.reshape(2 * B, D)

    idx = jnp.concatenate([2 * node_list[:, None], 2 * neigh_indices + 1],
                          axis=1)
    idx = jnp.concatenate([idx, jnp.zeros((BP - B, SEG), jnp.int32)], axis=0)
    idx3 = idx.reshape(NW, NGRP, SEG * G)

    return _gather_reduce(table, idx3)[:B]


# speedup vs baseline: 1.0316x; 1.0316x over previous
import functools

import jax
import jax.numpy as jnp
from jax import lax
from jax.experimental import pallas as pl
from jax.experimental.pallas import tpu as pltpu
from jax.experimental.pallas import tpu_sc as plsc

B = 50000
S = 10
D = 128
SEG = S + 1
G = 8
NW = 32
NGRP = 196
PER_W = NGRP * G
BP = NW * PER_W
L = 16
NBUF = 4
NITER = NGRP // NBUF

_mesh = plsc.VectorSubcoreMesh(core_axis_name="c", subcore_axis_name="s")


def _mm_body(x_ref, w_ref, o_ref):
    o_ref[...] = jnp.dot(x_ref[...], w_ref[...],
                         preferred_element_type=jnp.float32)


def _transform_table(features, Wc):
    blk = 2000
    return pl.pallas_call(
        _mm_body,
        grid=(B // blk,),
        in_specs=[
            pl.BlockSpec((blk, D), lambda i: (i, 0)),
            pl.BlockSpec((D, 2 * D), lambda i: (0, 0)),
        ],
        out_specs=pl.BlockSpec((blk, 2 * D), lambda i: (i, 0)),
        out_shape=jax.ShapeDtypeStruct((B, 2 * D), jnp.float32),
    )(features, Wc)


def _reduce_group(rows_v, out_v):
    for dd in range(G):
        for j in range(D // L):
            sl = pl.ds(j * L, L)
            acc = rows_v[dd * SEG, sl]
            for t in range(1, SEG):
                acc = acc + rows_v[dd * SEG + t, sl]
            out_v[dd, sl] = acc


@functools.partial(
    pl.kernel,
    mesh=_mesh,
    out_type=jax.ShapeDtypeStruct((B, D), jnp.float32),
    scratch_types=[
        pltpu.VMEM((NGRP, SEG * G), jnp.int32),
        pltpu.VMEM((G, D), jnp.float32),
    ] + [pltpu.VMEM((SEG * G, D), jnp.float32) for _ in range(NBUF)]
      + [pltpu.SemaphoreType.DMA for _ in range(NBUF)],
)
def _gather_reduce(table_hbm, idx_hbm, out_hbm, idx_v, out_v, *bufsem):
    bufs, sems = bufsem[:NBUF], bufsem[NBUF:]
    wid = lax.axis_index("s") * 2 + lax.axis_index("c")
    base = wid * PER_W
    n_out = jnp.clip((B - base) // G, 0, NGRP)

    pltpu.sync_copy(idx_hbm.at[wid], idx_v)
    for l in range(NBUF):
        pltpu.async_copy(table_hbm.at[idx_v.at[l]], bufs[l], sems[l])

    def body(gq, carry):
        for l in range(NBUF):
            g = NBUF * gq + l
            pltpu.make_async_copy(table_hbm.at[idx_v.at[0]],
                                  bufs[l], sems[l]).wait()
            _reduce_group(bufs[l], out_v)

            @pl.when(g < n_out)
            def _():
                pltpu.sync_copy(out_v, out_hbm.at[pl.ds(base + g * G, G)])

            @pl.when(gq < NITER - 1)
            def _():
                pltpu.async_copy(table_hbm.at[idx_v.at[g + NBUF]],
                                 bufs[l], sems[l])
        return carry

    lax.fori_loop(0, NITER, body, 0)


def kernel(features, node_list, neigh_indices, W):
    Wc = jnp.concatenate([W[:, :D].T, W[:, D:].T * jnp.float32(1.0 / S)],
                         axis=1)
    ft = _transform_table(features, Wc)
    table = ft.reshape(2 * B, D)

    idx = jnp.concatenate([2 * node_list[:, None], 2 * neigh_indices + 1],
                          axis=1)
    idx = jnp.concatenate([idx, jnp.zeros((BP - B, SEG), jnp.int32)], axis=0)
    idx3 = idx.reshape(NW, NGRP, SEG * G)

    return _gather_reduce(table, idx3)


# speedup vs baseline: 1.1647x; 1.1290x over previous
import functools

import jax
import jax.numpy as jnp
from jax import lax
from jax.experimental import pallas as pl
from jax.experimental.pallas import tpu as pltpu
from jax.experimental.pallas import tpu_sc as plsc

B = 50000
S = 10
D = 128
SEG = S + 1
G = 8
NW = 32
NGRP = 196
PER_W = NGRP * G
BP = NW * PER_W
L = 16
NBUF = 4
NITER = NGRP // NBUF

_mesh = plsc.VectorSubcoreMesh(core_axis_name="c", subcore_axis_name="s")


def _mm_body(x_ref, w_ref, o_ref):
    y = jnp.dot(x_ref[...], w_ref[...], preferred_element_type=jnp.float32)
    o_ref[...] = y.reshape(o_ref.shape)


def _transform_table(features, Wc):
    blk = 2000
    return pl.pallas_call(
        _mm_body,
        grid=(B // blk,),
        in_specs=[
            pl.BlockSpec((blk, D), lambda i: (i, 0)),
            pl.BlockSpec((D, 2 * D), lambda i: (0, 0)),
        ],
        out_specs=pl.BlockSpec((2 * blk, D), lambda i: (i, 0)),
        out_shape=jax.ShapeDtypeStruct((2 * B, D), jnp.float32),
    )(features, Wc)


def _reduce_group(rows_v, out_v):
    for dd in range(G):
        for j in range(D // L):
            sl = pl.ds(j * L, L)
            acc = rows_v[dd * SEG, sl]
            for t in range(1, SEG):
                acc = acc + rows_v[dd * SEG + t, sl]
            out_v[dd, sl] = acc


@functools.partial(
    pl.kernel,
    mesh=_mesh,
    out_type=jax.ShapeDtypeStruct((B, D), jnp.float32),
    scratch_types=[
        pltpu.VMEM((NGRP, SEG * G), jnp.int32),
        pltpu.VMEM((G, D), jnp.float32),
    ] + [pltpu.VMEM((SEG * G, D), jnp.float32) for _ in range(NBUF)]
      + [pltpu.SemaphoreType.DMA for _ in range(NBUF)],
)
def _gather_reduce(table_hbm, idx_hbm, out_hbm, idx_v, out_v, *bufsem):
    bufs, sems = bufsem[:NBUF], bufsem[NBUF:]
    wid = lax.axis_index("s") * 2 + lax.axis_index("c")
    base = wid * PER_W
    n_out = jnp.clip((B - base) // G, 0, NGRP)

    pltpu.sync_copy(idx_hbm.at[wid], idx_v)
    for l in range(NBUF):
        pltpu.async_copy(table_hbm.at[idx_v.at[l]], bufs[l], sems[l])

    def body(gq, carry):
        for l in range(NBUF):
            g = NBUF * gq + l
            pltpu.make_async_copy(table_hbm.at[idx_v.at[0]],
                                  bufs[l], sems[l]).wait()
            _reduce_group(bufs[l], out_v)

            @pl.when(g < n_out)
            def _():
                pltpu.sync_copy(out_v, out_hbm.at[pl.ds(base + g * G, G)])

            @pl.when(gq < NITER - 1)
            def _():
                pltpu.async_copy(table_hbm.at[idx_v.at[g + NBUF]],
                                 bufs[l], sems[l])
        return carry

    lax.fori_loop(0, NITER, body, 0)


def kernel(features, node_list, neigh_indices, W):
    Wc = jnp.concatenate([W[:, :D].T, W[:, D:].T * jnp.float32(1.0 / S)],
                         axis=1)
    table = _transform_table(features, Wc)

    idx = jnp.concatenate([2 * node_list[:, None], 2 * neigh_indices + 1],
                          axis=1)
    idx = jnp.concatenate([idx, jnp.zeros((BP - B, SEG), jnp.int32)], axis=0)
    idx3 = idx.reshape(NW, NGRP, SEG * G)

    return _gather_reduce(table, idx3)
